# Initial kernel scaffold; baseline (speedup 1.0000x reference)
import jax
import jax.numpy as jnp
from jax.experimental import pallas as pl


def kernel(nfeats, edge_index, efeats, Wm1, bm1, Wa1, ba1, Wm2, bm2, Wa2, ba2, Wm3, bm3, Wa3, ba3):
    raise NotImplementedError("write your pallas kernel here")


# speedup vs baseline: 2.1323x; 2.1323x over previous
import functools

import jax
import jax.numpy as jnp
from jax import lax
from jax.experimental import pallas as pl
from jax.experimental.pallas import tpu as pltpu
from jax.experimental.pallas import tpu_sc as plsc

N = 10000
E = 320000
NP = 10240
EP = 327680
K = 128
NC = 2
NS = 16
NW = NC * NS
E_PER_W = EP // NW
N_CHUNKS = E_PER_W // K
ROWS_PER_TILE = NP // NS


def _spmm_sc(table, src, dst, zeros, d):
    mesh = plsc.VectorSubcoreMesh(core_axis_name="c", subcore_axis_name="s")

    @functools.partial(
        pl.kernel,
        out_type=jax.ShapeDtypeStruct((NC, NP, d), jnp.float32),
        mesh=mesh,
        scratch_types=[
            pltpu.VMEM((K,), jnp.int32),
            pltpu.VMEM((K,), jnp.int32),
            pltpu.VMEM((K, d), jnp.float32),
            pltpu.VMEM_SHARED((NP, d), jnp.float32),
            pltpu.SemaphoreType.DMA,
        ],
        compiler_params=pltpu.CompilerParams(use_tc_tiling_on_sc=False),
    )
    def spmm(table_hbm, src_hbm, dst_hbm, zeros_hbm, out_hbm,
             src_v, dst_v, rows_v, acc_sh, sem):
        cid = lax.axis_index("c")
        sid = lax.axis_index("s")
        wid = sid * NC + cid
        r0 = sid * ROWS_PER_TILE
        pltpu.sync_copy(zeros_hbm.at[pl.ds(r0, ROWS_PER_TILE)],
                        acc_sh.at[pl.ds(r0, ROWS_PER_TILE)])
        plsc.subcore_barrier()

        ebase = wid * E_PER_W

        def body(j, carry):
            base = ebase + j * K
            pltpu.sync_copy(src_hbm.at[pl.ds(base, K)], src_v)
            pltpu.sync_copy(dst_hbm.at[pl.ds(base, K)], dst_v)
            pltpu.async_copy(table_hbm.at[src_v], rows_v, sem).wait()
            pltpu.sync_copy(rows_v, acc_sh.at[dst_v], add=True)
            return carry

        lax.fori_loop(0, N_CHUNKS, body, 0)
        plsc.subcore_barrier()
        pltpu.sync_copy(acc_sh.at[pl.ds(r0, ROWS_PER_TILE)],
                        out_hbm.at[cid, pl.ds(r0, ROWS_PER_TILE)])

    return spmm(table, src, dst, zeros)


BLK = 512


def _dense_layer(h, gp, efap, wmt, wme, bm, wat, wan, ba):
    din = h.shape[1]
    dm = wmt.shape[1]

    def body(h_ref, g_ref, efa_ref, wmt_ref, wme_ref, bm_ref,
             wat_ref, wan_ref, ba_ref, out_ref):
        g = g_ref[0] + g_ref[1]
        efa = efa_ref[0] + efa_ref[1]
        cnt = efa[:, 16:17]
        inv = 1.0 / jnp.maximum(cnt, 1.0)
        s = (jnp.dot(g, wmt_ref[...], preferred_element_type=jnp.float32)
             + jnp.dot(efa[:, :16], wme_ref[...],
                       preferred_element_type=jnp.float32)
             + cnt * bm_ref[...])
        hn = s * inv
        out = (jnp.dot(h_ref[...], wat_ref[...],
                       preferred_element_type=jnp.float32)
               + jnp.dot(hn, wan_ref[...], preferred_element_type=jnp.float32)
               + ba_ref[...])
        out_ref[...] = jnp.maximum(out, 0.0)

    grid = NP // BLK
    return pl.pallas_call(
        body,
        grid=(grid,),
        in_specs=[
            pl.BlockSpec((BLK, din), lambda i: (i, 0)),
            pl.BlockSpec((NC, BLK, din), lambda i: (0, i, 0)),
            pl.BlockSpec((NC, BLK, 32), lambda i: (0, i, 0)),
            pl.BlockSpec((din, dm), lambda i: (0, 0)),
            pl.BlockSpec((16, dm), lambda i: (0, 0)),
            pl.BlockSpec((1, dm), lambda i: (0, 0)),
            pl.BlockSpec((din, dm), lambda i: (0, 0)),
            pl.BlockSpec((dm, dm), lambda i: (0, 0)),
            pl.BlockSpec((1, dm), lambda i: (0, 0)),
        ],
        out_specs=pl.BlockSpec((BLK, dm), lambda i: (i, 0)),
        out_shape=jax.ShapeDtypeStruct((NP, dm), jnp.float32),
    )(h, gp, efap, wmt, wme, bm, wat, wan, ba)


def _pad2(w, r, c):
    return jnp.pad(w, ((0, r - w.shape[0]), (0, c - w.shape[1])))


def kernel(nfeats, edge_index, efeats, Wm1, bm1, Wa1, ba1,
           Wm2, bm2, Wa2, ba2, Wm3, bm3, Wa3, ba3):
    f32 = jnp.float32
    h0 = jnp.pad(nfeats[:, 0, :], ((0, NP - N), (0, 0)))
    src = jnp.pad(edge_index[0], (0, EP - E))
    dst = jnp.pad(edge_index[1], (0, EP - E), constant_values=N)
    ef_aug = jnp.zeros((EP, 32), f32)
    ef_aug = ef_aug.at[:E, :16].set(efeats[:, 0, :])
    ef_aug = ef_aug.at[:E, 16].set(1.0)
    eidx = jnp.arange(EP, dtype=jnp.int32)

    z160 = jnp.zeros((NP, 160), f32)
    z128 = jnp.zeros((NP, 128), f32)
    z32 = jnp.zeros((NP, 32), f32)

    efap = _spmm_sc(ef_aug, eidx, dst, z32, 32)

    DH, DIN, DOUT, DHP = 152, 128, 128, 160
    g1 = _spmm_sc(h0, src, dst, z128, DIN)
    h1 = _dense_layer(
        h0, g1, efap,
        _pad2(Wm1[:DIN], DIN, DHP), _pad2(Wm1[DIN:], 16, DHP),
        _pad2(bm1[None, :], 1, DHP),
        _pad2(Wa1[:DIN], DIN, DHP), _pad2(Wa1[DIN:], DHP, DHP),
        _pad2(ba1[None, :], 1, DHP))

    g2 = _spmm_sc(h1, src, dst, z160, DHP)
    h2 = _dense_layer(
        h1, g2, efap,
        _pad2(Wm2[:DH], DHP, DHP), _pad2(Wm2[DH:], 16, DHP),
        _pad2(bm2[None, :], 1, DHP),
        _pad2(Wa2[:DH], DHP, DHP), _pad2(Wa2[DH:], DHP, DHP),
        _pad2(ba2[None, :], 1, DHP))

    g3 = _spmm_sc(h2, src, dst, z160, DHP)
    h3 = _dense_layer(
        h2, g3, efap,
        _pad2(Wm3[:DH], DHP, DOUT), _pad2(Wm3[DH:], 16, DOUT),
        bm3[None, :],
        _pad2(Wa3[:DH], DHP, DOUT), _pad2(Wa3[DH:], DOUT, DOUT),
        ba3[None, :])

    return h3[:N]


# speedup vs baseline: 2.6066x; 1.2224x over previous
import functools

import jax
import jax.numpy as jnp
from jax import lax
from jax.experimental import pallas as pl
from jax.experimental.pallas import tpu as pltpu
from jax.experimental.pallas import tpu_sc as plsc

N = 10000
E = 320000
NP = 10240
EP = 327680
K = 128
NC = 2
NS = 16
NW = NC * NS
E_PER_W = EP // NW
N_CHUNKS = E_PER_W // K
NBUF = 4
GROUPS = N_CHUNKS // NBUF
ROWS_PER_TILE = NP // NS


def _spmm_sc(tlo, thi, src3, dst3, zeros, dh):
    mesh = plsc.VectorSubcoreMesh(core_axis_name="c", subcore_axis_name="s")

    @functools.partial(
        pl.kernel,
        out_type=jax.ShapeDtypeStruct((2, NC, NP, dh), jnp.float32),
        mesh=mesh,
        scratch_types=[
            pltpu.VMEM((N_CHUNKS, K), jnp.int32),
            pltpu.VMEM((N_CHUNKS, K), jnp.int32),
            pltpu.VMEM((NBUF, K, dh), jnp.float32),
            pltpu.VMEM_SHARED((NP, dh), jnp.float32),
        ]
        + [pltpu.SemaphoreType.DMA] * (2 * NBUF),
        compiler_params=pltpu.CompilerParams(use_tc_tiling_on_sc=False),
    )
    def spmm(tlo_hbm, thi_hbm, src_hbm, dst_hbm, zeros_hbm, out_hbm,
             srcs_v, dsts_v, rows_v, acc_sh, *sems):
        gsem = sems[:NBUF]
        ssem = sems[NBUF:]
        cid = lax.axis_index("c")
        sid = lax.axis_index("s")
        wid = sid * NC + cid
        r0 = sid * ROWS_PER_TILE
        pltpu.sync_copy(src_hbm.at[wid], srcs_v)
        pltpu.sync_copy(dst_hbm.at[wid], dsts_v)

        def wait_gather(b):
            pltpu.make_async_copy(zeros_hbm.at[pl.ds(0, K)],
                                  rows_v.at[b], gsem[b]).wait()

        def wait_scatter(b):
            pltpu.make_async_copy(rows_v.at[b],
                                  acc_sh.at[pl.ds(0, K)], ssem[b]).wait()

        for half, tab in enumerate((tlo_hbm, thi_hbm)):
            pltpu.sync_copy(zeros_hbm.at[pl.ds(r0, ROWS_PER_TILE)],
                            acc_sh.at[pl.ds(r0, ROWS_PER_TILE)])
            plsc.subcore_barrier()

            for b in range(NBUF):
                pltpu.async_copy(tab.at[srcs_v.at[b]], rows_v.at[b],
                                 gsem[b])

            @pl.loop(0, GROUPS)
            def grp(g):
                for b in range(NBUF):
                    wait_gather(b)
                    pltpu.async_copy(rows_v.at[b],
                                     acc_sh.at[dsts_v.at[g * NBUF + b]],
                                     ssem[b], add=True)

                @pl.when(g < GROUPS - 1)
                def _():
                    for b in range(NBUF):
                        wait_scatter(b)
                        pltpu.async_copy(
                            tab.at[srcs_v.at[(g + 1) * NBUF + b]],
                            rows_v.at[b], gsem[b])

            for b in range(NBUF):
                wait_scatter(b)
            plsc.subcore_barrier()
            pltpu.sync_copy(acc_sh.at[pl.ds(r0, ROWS_PER_TILE)],
                            out_hbm.at[half, cid, pl.ds(r0, ROWS_PER_TILE)])

    return spmm(tlo, thi, src3, dst3, zeros)


BLK = 512


def _dense_layer(h, gp, efap, wmt, wme, bm, wat, wan, ba):
    din = h.shape[1]
    dh = din // 2
    dm = wmt.shape[1]

    def body(h_ref, g_ref, efa_ref, wmt_ref, wme_ref, bm_ref,
             wat_ref, wan_ref, ba_ref, out_ref):
        g_lo = g_ref[0, 0] + g_ref[0, 1]
        g_hi = g_ref[1, 0] + g_ref[1, 1]
        efa = efa_ref[0, 0] + efa_ref[0, 1]
        cnt = (efa_ref[1, 0] + efa_ref[1, 1])[:, 0:1]
        inv = 1.0 / jnp.maximum(cnt, 1.0)
        s = (jnp.dot(g_lo, wmt_ref[:dh], preferred_element_type=jnp.float32)
             + jnp.dot(g_hi, wmt_ref[dh:],
                       preferred_element_type=jnp.float32)
             + jnp.dot(efa, wme_ref[...], preferred_element_type=jnp.float32)
             + cnt * bm_ref[...])
        hn = s * inv
        out = (jnp.dot(h_ref[...], wat_ref[...],
                       preferred_element_type=jnp.float32)
               + jnp.dot(hn, wan_ref[...], preferred_element_type=jnp.float32)
               + ba_ref[...])
        out_ref[...] = jnp.maximum(out, 0.0)

    grid = NP // BLK
    return pl.pallas_call(
        body,
        grid=(grid,),
        in_specs=[
            pl.BlockSpec((BLK, din), lambda i: (i, 0)),
            pl.BlockSpec((2, NC, BLK, dh), lambda i: (0, 0, i, 0)),
            pl.BlockSpec((2, NC, BLK, 16), lambda i: (0, 0, i, 0)),
            pl.BlockSpec((din, dm), lambda i: (0, 0)),
            pl.BlockSpec((16, dm), lambda i: (0, 0)),
            pl.BlockSpec((1, dm), lambda i: (0, 0)),
            pl.BlockSpec((din, dm), lambda i: (0, 0)),
            pl.BlockSpec((dm, dm), lambda i: (0, 0)),
            pl.BlockSpec((1, dm), lambda i: (0, 0)),
        ],
        out_specs=pl.BlockSpec((BLK, dm), lambda i: (i, 0)),
        out_shape=jax.ShapeDtypeStruct((NP, dm), jnp.float32),
    )(h, gp, efap, wmt, wme, bm, wat, wan, ba)


def _pad2(w, r, c):
    return jnp.pad(w, ((0, r - w.shape[0]), (0, c - w.shape[1])))


def kernel(nfeats, edge_index, efeats, Wm1, bm1, Wa1, ba1,
           Wm2, bm2, Wa2, ba2, Wm3, bm3, Wa3, ba3):
    f32 = jnp.float32
    h0 = jnp.pad(nfeats[:, 0, :], ((0, NP - N), (0, 0)))
    src3 = jnp.pad(edge_index[0], (0, EP - E)).reshape(NW, N_CHUNKS, K)
    dst3 = jnp.pad(edge_index[1], (0, EP - E),
                   constant_values=N).reshape(NW, N_CHUNKS, K)
    ef_lo = jnp.pad(efeats[:, 0, :], ((0, EP - E), (0, 0)))
    ef_hi = jnp.pad(jnp.ones((E, 1), f32), ((0, EP - E), (0, 15)))
    eidx3 = jnp.arange(EP, dtype=jnp.int32).reshape(NW, N_CHUNKS, K)

    z80 = jnp.zeros((NP, 80), f32)
    z64 = jnp.zeros((NP, 64), f32)
    z16 = jnp.zeros((NP, 16), f32)

    efap = _spmm_sc(ef_lo, ef_hi, eidx3, dst3, z16, 16)

    DH, DIN, DOUT, DHP = 152, 128, 128, 160
    g1 = _spmm_sc(h0[:, :64], h0[:, 64:], src3, dst3, z64, 64)
    h1 = _dense_layer(
        h0, g1, efap,
        _pad2(Wm1[:DIN], DIN, DHP), _pad2(Wm1[DIN:], 16, DHP),
        _pad2(bm1[None, :], 1, DHP),
        _pad2(Wa1[:DIN], DIN, DHP), _pad2(Wa1[DIN:], DHP, DHP),
        _pad2(ba1[None, :], 1, DHP))

    g2 = _spmm_sc(h1[:, :80], h1[:, 80:], src3, dst3, z80, 80)
    h2 = _dense_layer(
        h1, g2, efap,
        _pad2(Wm2[:DH], DHP, DHP), _pad2(Wm2[DH:], 16, DHP),
        _pad2(bm2[None, :], 1, DHP),
        _pad2(Wa2[:DH], DHP, DHP), _pad2(Wa2[DH:], DHP, DHP),
        _pad2(ba2[None, :], 1, DHP))

    g3 = _spmm_sc(h2[:, :80], h2[:, 80:], src3, dst3, z80, 80)
    h3 = _dense_layer(
        h2, g3, efap,
        _pad2(Wm3[:DH], DHP, DOUT), _pad2(Wm3[DH:], 16, DOUT),
        bm3[None, :],
        _pad2(Wa3[:DH], DHP, DOUT), _pad2(Wa3[DH:], DOUT, DOUT),
        ba3[None, :])

    return h3[:N]
